# baseline (speedup 1.0000x reference)
import functools

import jax
import jax.numpy as jnp
from jax import lax
from jax.experimental import pallas as pl
from jax.experimental.pallas import tpu as pltpu
from jax.experimental.pallas import tpu_sc as plsc

_EMBED = 32
_SEQ = 50
_NTOK = 16384
_TB = 128
_NC = 2
_NS = 16
_NW = _NC * _NS
_WTOK = _NTOK // _NW
_NTI = _WTOK // _TB
_NBUF = 2


def _sc_body(table_hbm, idx_hbm, out_hbm, idxv, rowsv, outv, isem, gsems, osems):
    wid = lax.axis_index("s") * _NC + lax.axis_index("c")
    tw = wid * _NTI

    pltpu.sync_copy(idx_hbm.at[:, pl.ds(wid * _WTOK, _WTOK)], idxv)

    def gathers(s, b):
        return [
            pltpu.make_async_copy(
                table_hbm.at[idxv.at[s, pl.ds(ti * _TB, _TB)]],
                rowsv[b].at[pl.ds(ti * _TB, _TB)],
                gsems[b],
            )
            for ti in range(_NTI)
        ]

    def out_copy(s, b):
        return pltpu.make_async_copy(
            outv[b], out_hbm.at[s].at[:, pl.ds(tw, _NTI)], osems[b]
        )

    iota = lax.iota(jnp.int32, 16)
    e8 = [(iota + 16 * h) >> 3 for h in range(2)]
    e8i = [(iota + 16 * h) & 7 for h in range(2)]

    def transpose(b):
        for ti in range(_NTI):
            tiv = jnp.full((16,), ti, jnp.int32)

            @plsc.parallel_loop(0, _TB, 1, unroll=16)
            def _(tl):
                t = ti * _TB + tl
                tlv = jnp.full((16,), tl, jnp.int32)
                v0 = rowsv[b][t, pl.ds(0, 16)]
                v1 = rowsv[b][t, pl.ds(16, 16)]
                plsc.store_scatter(outv[b], [e8[0], tiv, e8i[0], tlv], v0)
                plsc.store_scatter(outv[b], [e8[1], tiv, e8i[1], tlv], v1)

    def step(s, b, issue_gather, wait_out):
        for cp in gathers(s, b):
            cp.wait()
        if wait_out:
            out_copy(s - _NBUF, b).wait()
        transpose(b)
        out_copy(s, b).start()
        if issue_gather:
            for cp in gathers(s + _NBUF, b):
                cp.start()

    for b in range(_NBUF):
        for cp in gathers(b, b):
            cp.start()

    for s in range(_NBUF):
        step(s, s, True, False)

    @pl.loop(_NBUF, _SEQ - _NBUF, step=_NBUF)
    def _(ss):
        for u in range(_NBUF):
            step(ss + u, u, True, True)

    for s in range(_SEQ - _NBUF, _SEQ):
        step(s, s % _NBUF, False, True)
    for s in range(_SEQ - _NBUF, _SEQ):
        out_copy(s, s % _NBUF).wait()


_sc_gather = functools.partial(
    pl.kernel,
    out_type=jax.ShapeDtypeStruct((_SEQ, 4, _TB, 8, _TB), jnp.float32),
    mesh=plsc.VectorSubcoreMesh(core_axis_name="c", subcore_axis_name="s"),
    compiler_params=pltpu.CompilerParams(
        use_tc_tiling_on_sc=False, needs_layout_passes=False
    ),
    scratch_types=[
        pltpu.VMEM((_SEQ, _WTOK), jnp.int32),
        [pltpu.VMEM((_WTOK, _EMBED), jnp.float32) for _ in range(_NBUF)],
        [pltpu.VMEM((4, _NTI, 8, _TB), jnp.float32) for _ in range(_NBUF)],
        pltpu.SemaphoreType.DMA,
        [pltpu.SemaphoreType.DMA for _ in range(_NBUF)],
        [pltpu.SemaphoreType.DMA for _ in range(_NBUF)],
    ],
)(_sc_body)


@jax.jit
def kernel(tokenid, table):
    idx_t = jnp.swapaxes(tokenid, 0, 1)
    out5 = _sc_gather(table, idx_t)
    return out5.transpose(2, 4, 0, 1, 3).reshape(_NTOK, _SEQ, _EMBED)


# speedup vs baseline: 1.5642x; 1.5642x over previous
import functools

import jax
import jax.numpy as jnp
from jax import lax
from jax.experimental import pallas as pl
from jax.experimental.pallas import tpu as pltpu
from jax.experimental.pallas import tpu_sc as plsc

_EMBED = 32
_SEQ = 50
_NTOK = 16384
_TB = 128
_NC = 2
_NS = 16
_NW = _NC * _NS
_WTOK = _NTOK // _NW
_NTI = _WTOK // _TB
_NBUF = 2


def _sc_body(table_hbm, idx_hbm, out_hbm, idxv, rowsv, outv, isem, gsems, osems):
    wid = lax.axis_index("s") * _NC + lax.axis_index("c")
    tw = wid * _NTI

    pltpu.sync_copy(idx_hbm.at[:, pl.ds(wid * _WTOK, _WTOK)], idxv)

    def gathers(s, b):
        return [
            pltpu.make_async_copy(
                table_hbm.at[idxv.at[s, pl.ds(ti * _TB, _TB)]],
                rowsv[b].at[pl.ds(ti * _TB, _TB)],
                gsems[b],
            )
            for ti in range(_NTI)
        ]

    def out_copies(s, b):
        return [
            pltpu.make_async_copy(
                outv[b].at[ti, :, :, pl.ds(0, _TB)],
                out_hbm.at[s].at[:, tw + ti],
                osems[b],
            )
            for ti in range(_NTI)
        ]

    iota = lax.iota(jnp.int32, 16)
    e8 = [(iota + 16 * h) >> 3 for h in range(2)]
    e8i = [(iota + 16 * h) & 7 for h in range(2)]

    def transpose(b):
        @plsc.parallel_loop(0, _WTOK, 1, unroll=8)
        def _(t):
            tiv = jnp.full((16,), t >> 7, jnp.int32)
            tlv = jnp.full((16,), t & 127, jnp.int32)
            v0 = rowsv[b][t, pl.ds(0, 16)]
            v1 = rowsv[b][t, pl.ds(16, 16)]
            plsc.store_scatter(outv[b], [tiv, e8[0], e8i[0], tlv], v0)
            plsc.store_scatter(outv[b], [tiv, e8[1], e8i[1], tlv], v1)

    def step(s, b, issue_gather, wait_out):
        for cp in gathers(s, b):
            cp.wait()
        if wait_out:
            for cp in out_copies(s - _NBUF, b):
                cp.wait()
        transpose(b)
        for cp in out_copies(s, b):
            cp.start()
        if issue_gather:
            for cp in gathers(s + _NBUF, b):
                cp.start()

    for b in range(_NBUF):
        for cp in gathers(b, b):
            cp.start()

    for s in range(_NBUF):
        step(s, s, True, False)

    @pl.loop(_NBUF, _SEQ - _NBUF, step=_NBUF)
    def _(ss):
        for u in range(_NBUF):
            step(ss + u, u, True, True)

    for s in range(_SEQ - _NBUF, _SEQ):
        step(s, s % _NBUF, False, True)
    for s in range(_SEQ - _NBUF, _SEQ):
        for cp in out_copies(s, s % _NBUF):
            cp.wait()


_sc_gather = functools.partial(
    pl.kernel,
    out_type=jax.ShapeDtypeStruct((_SEQ, 4, _TB, 8, _TB), jnp.float32),
    mesh=plsc.VectorSubcoreMesh(core_axis_name="c", subcore_axis_name="s"),
    compiler_params=pltpu.CompilerParams(
        use_tc_tiling_on_sc=False, needs_layout_passes=False
    ),
    scratch_types=[
        pltpu.VMEM((_SEQ, _WTOK), jnp.int32),
        [pltpu.VMEM((_WTOK, _EMBED), jnp.float32) for _ in range(_NBUF)],
        [pltpu.VMEM((_NTI, 4, 8, _TB + 1), jnp.float32) for _ in range(_NBUF)],
        pltpu.SemaphoreType.DMA,
        [pltpu.SemaphoreType.DMA for _ in range(_NBUF)],
        [pltpu.SemaphoreType.DMA for _ in range(_NBUF)],
    ],
)(_sc_body)


@jax.jit
def kernel(tokenid, table):
    idx_t = jnp.swapaxes(tokenid, 0, 1)
    out5 = _sc_gather(table, idx_t)
    return out5.transpose(2, 4, 0, 1, 3).reshape(_NTOK, _SEQ, _EMBED)


# speedup vs baseline: 1.6950x; 1.0836x over previous
import functools

import jax
import jax.numpy as jnp
from jax import lax
from jax.experimental import pallas as pl
from jax.experimental.pallas import tpu as pltpu
from jax.experimental.pallas import tpu_sc as plsc

_EMBED = 32
_SEQ = 50
_NTOK = 16384
_TB = 128
_NC = 2
_NS = 16
_NW = _NC * _NS
_WTOK = _NTOK // _NW
_NTI = _WTOK // _TB
_NBUF = 2


def _sc_body(table_hbm, idx_hbm, out_hbm, idxv, rowsv, outv, isem, gsems, osems):
    wid = lax.axis_index("s") * _NC + lax.axis_index("c")
    tw = wid * _NTI

    pltpu.sync_copy(idx_hbm.at[:, pl.ds(wid * _WTOK, _WTOK)], idxv)

    def gathers(s, b):
        return [
            pltpu.make_async_copy(
                table_hbm.at[idxv.at[s, pl.ds(ti * _TB, _TB)]],
                rowsv[b].at[pl.ds(ti * _TB, _TB)],
                gsems[b],
            )
            for ti in range(_NTI)
        ]

    def out_copies(s, b):
        return [
            pltpu.make_async_copy(
                outv[b].at[ti, :, :, pl.ds(0, _TB)],
                out_hbm.at[s].at[:, tw + ti],
                osems[b],
            )
            for ti in range(_NTI)
        ]

    iota = lax.iota(jnp.int32, 16)
    e8 = [(iota + 16 * h) >> 3 for h in range(2)]
    e8i = [(iota + 16 * h) & 7 for h in range(2)]

    def transpose(b):
        @plsc.parallel_loop(0, _WTOK, 1, unroll=8)
        def _(t):
            tiv = jnp.full((16,), t >> 7, jnp.int32)
            tlv = jnp.full((16,), t & 127, jnp.int32)
            v0 = rowsv[b][t, pl.ds(0, 16)]
            v1 = rowsv[b][t, pl.ds(16, 16)]
            plsc.store_scatter(outv[b], [tiv, e8[0], e8i[0], tlv], v0)
            plsc.store_scatter(outv[b], [tiv, e8[1], e8i[1], tlv], v1)

    def step(s, b, issue_gather, wait_out):
        for cp in gathers(s, b):
            cp.wait()
        if wait_out:
            for cp in out_copies(s - _NBUF, b):
                cp.wait()
        transpose(b)
        for cp in out_copies(s, b):
            cp.start()
        if issue_gather:
            for cp in gathers(s + _NBUF, b):
                cp.start()

    for b in range(_NBUF):
        for cp in gathers(b, b):
            cp.start()

    for s in range(_NBUF):
        step(s, s, True, False)

    @pl.loop(_NBUF, _SEQ - _NBUF, step=_NBUF)
    def _(ss):
        for u in range(_NBUF):
            step(ss + u, u, True, True)

    for s in range(_SEQ - _NBUF, _SEQ):
        step(s, s % _NBUF, False, True)
    for s in range(_SEQ - _NBUF, _SEQ):
        for cp in out_copies(s, s % _NBUF):
            cp.wait()


_sc_gather = functools.partial(
    pl.kernel,
    out_type=jax.ShapeDtypeStruct((_SEQ, 4, _TB, 8, _TB), jnp.float32),
    mesh=plsc.VectorSubcoreMesh(core_axis_name="c", subcore_axis_name="s"),
    compiler_params=pltpu.CompilerParams(
        use_tc_tiling_on_sc=False, needs_layout_passes=False
    ),
    scratch_types=[
        pltpu.VMEM((_SEQ, _WTOK), jnp.int32),
        [pltpu.VMEM((_WTOK, _EMBED), jnp.float32) for _ in range(_NBUF)],
        [pltpu.VMEM((_NTI, 4, 8, _TB + 1), jnp.float32) for _ in range(_NBUF)],
        pltpu.SemaphoreType.DMA,
        [pltpu.SemaphoreType.DMA for _ in range(_NBUF)],
        [pltpu.SemaphoreType.DMA for _ in range(_NBUF)],
    ],
)(_sc_body)


_VOCAB = 1000000
_CB = 128
_NFULL = _VOCAB // _CB
_TAILR = _VOCAB - _NFULL * _CB
_CPW = _NFULL // _NW
_K1BUF = 2


def _k1_body(tab_hbm, lin_hbm, inv, outv, tailv, gsems, osems):
    wid = lax.axis_index("s") * _NC + lax.axis_index("c")

    iota = lax.iota(jnp.int32, 16)
    rk = [(iota + 16 * k) >> 2 for k in range(8)]
    ck = [((iota + 16 * k) & 3) * _EMBED for k in range(8)]

    def in_copy(c, b):
        return pltpu.make_async_copy(
            tab_hbm.at[:, pl.ds(c * _CB, _CB)], inv[b], gsems[b]
        )

    def out_copy(c, b):
        return pltpu.make_async_copy(
            outv[b], lin_hbm.at[pl.ds(c * 32, 32)], osems[b]
        )

    def transpose_ref(src, b, ncol):
        @plsc.parallel_loop(0, _EMBED, 1, unroll=4)
        def _(e):
            ev = jnp.full((16,), e, jnp.int32)
            for k in range(ncol // 16):
                v = src[e, pl.ds(16 * k, 16)]
                plsc.store_scatter(outv[b], [rk[k], ck[k] + ev], v)

    def transpose(b, ncol):
        transpose_ref(inv[b], b, ncol)

    def blk(j):
        return wid + j * _NW

    for b in range(_K1BUF):
        in_copy(blk(b), b).start()

    def step(j, b, issue, wait_out):
        c = blk(j)
        in_copy(c, b).wait()
        if wait_out:
            out_copy(blk(j - _K1BUF), b).wait()
        transpose(b, _CB)
        out_copy(c, b).start()
        if issue:
            in_copy(blk(j + _K1BUF), b).start()

    for j in range(_K1BUF):
        step(j, j, True, False)

    @pl.loop(_K1BUF, _CPW - _K1BUF, step=_K1BUF)
    def _(jj):
        for u in range(_K1BUF):
            step(jj + u, u, True, True)

    for j in range(_CPW - _K1BUF, _CPW):
        step(j, j % _K1BUF, False, True)
    for j in range(_CPW - _K1BUF, _CPW):
        out_copy(blk(j), j % _K1BUF).wait()

    @pl.when(wid < 4)
    def _():
        c = _CPW * _NW + wid
        in_copy(c, 0).start()
        in_copy(c, 0).wait()
        transpose(0, _CB)
        cp = out_copy(c, 0)
        cp.start()
        cp.wait()

    @pl.when(wid == 4)
    def _():
        cp = pltpu.make_async_copy(
            tab_hbm.at[:, pl.ds(_NFULL * _CB, _TAILR)], tailv, gsems[0]
        )
        cp.start()
        cp.wait()
        transpose_ref(tailv, 0, _TAILR)
        cp = pltpu.make_async_copy(
            outv[0].at[pl.ds(0, _TAILR // 4)],
            lin_hbm.at[pl.ds(_NFULL * 32, _TAILR // 4)],
            osems[0],
        )
        cp.start()
        cp.wait()


_k1_detile = functools.partial(
    pl.kernel,
    out_type=jax.ShapeDtypeStruct((_VOCAB // 4, _CB), jnp.float32),
    mesh=plsc.VectorSubcoreMesh(core_axis_name="c", subcore_axis_name="s"),
    compiler_params=pltpu.CompilerParams(
        use_tc_tiling_on_sc=True, needs_layout_passes=False
    ),
    scratch_types=[
        [pltpu.VMEM((_EMBED, _CB), jnp.float32) for _ in range(_K1BUF)],
        [pltpu.VMEM((_EMBED, _CB), jnp.float32) for _ in range(_K1BUF)],
        pltpu.VMEM((_EMBED, _TAILR), jnp.float32),
        [pltpu.SemaphoreType.DMA for _ in range(_K1BUF)],
        [pltpu.SemaphoreType.DMA for _ in range(_K1BUF)],
    ],
)(_k1_body)


@jax.jit
def kernel(tokenid, table):
    idx_t = jnp.swapaxes(tokenid, 0, 1)
    lin = _k1_detile(jnp.swapaxes(table, 0, 1))
    table_lin = lin.reshape(_VOCAB, _EMBED)
    out5 = _sc_gather(table_lin, idx_t)
    return out5.transpose(2, 4, 0, 1, 3).reshape(_NTOK, _SEQ, _EMBED)


# speedup vs baseline: 2.3304x; 1.3748x over previous
import functools

import jax
import jax.numpy as jnp
from jax import lax
from jax.experimental import pallas as pl
from jax.experimental.pallas import tpu as pltpu
from jax.experimental.pallas import tpu_sc as plsc

_EMBED = 32
_SEQ = 50
_NTOK = 16384
_TB = 128
_NC = 2
_NS = 16
_NW = _NC * _NS
_WTOK = _NTOK // _NW
_NTI = _WTOK // _TB
_NBUF = 2


def _sc_body(table_hbm, idx_hbm, out_hbm, idxv, rowsv, outv, isem, gsems, osems):
    wid = lax.axis_index("s") * _NC + lax.axis_index("c")
    tw = wid * _NTI

    pltpu.sync_copy(idx_hbm.at[:, pl.ds(wid * _WTOK, _WTOK)], idxv)

    def gathers(s, b):
        return [
            pltpu.make_async_copy(
                table_hbm.at[idxv.at[s, pl.ds(ti * _TB, _TB)]],
                rowsv[b].at[pl.ds(ti * _TB, _TB)],
                gsems[b],
            )
            for ti in range(_NTI)
        ]

    def out_copies(s, b):
        return [
            pltpu.make_async_copy(
                outv[b].at[ti, :, :, pl.ds(0, _TB)],
                out_hbm.at[s].at[:, tw + ti],
                osems[b],
            )
            for ti in range(_NTI)
        ]

    iota = lax.iota(jnp.int32, 16)
    e8 = [(iota + 16 * h) >> 3 for h in range(2)]
    e8i = [(iota + 16 * h) & 7 for h in range(2)]

    def transpose(b):
        @plsc.parallel_loop(0, _WTOK, 1, unroll=8)
        def _(t):
            tiv = jnp.full((16,), t >> 7, jnp.int32)
            tlv = jnp.full((16,), t & 127, jnp.int32)
            v0 = rowsv[b][t, pl.ds(0, 16)]
            v1 = rowsv[b][t, pl.ds(16, 16)]
            plsc.store_scatter(outv[b], [tiv, e8[0], e8i[0], tlv], v0)
            plsc.store_scatter(outv[b], [tiv, e8[1], e8i[1], tlv], v1)

    def step(s, b, issue_gather, wait_out):
        for cp in gathers(s, b):
            cp.wait()
        if wait_out:
            for cp in out_copies(s - _NBUF, b):
                cp.wait()
        transpose(b)
        for cp in out_copies(s, b):
            cp.start()
        if issue_gather:
            for cp in gathers(s + _NBUF, b):
                cp.start()

    for b in range(_NBUF):
        for cp in gathers(b, b):
            cp.start()

    for s in range(_NBUF):
        step(s, s, True, False)

    @pl.loop(_NBUF, _SEQ - _NBUF, step=_NBUF)
    def _(ss):
        for u in range(_NBUF):
            step(ss + u, u, True, True)

    for s in range(_SEQ - _NBUF, _SEQ):
        step(s, s % _NBUF, False, True)
    for s in range(_SEQ - _NBUF, _SEQ):
        for cp in out_copies(s, s % _NBUF):
            cp.wait()


_sc_gather = functools.partial(
    pl.kernel,
    out_type=jax.ShapeDtypeStruct((_SEQ, 4, _TB, 8, _TB), jnp.float32),
    mesh=plsc.VectorSubcoreMesh(core_axis_name="c", subcore_axis_name="s"),
    compiler_params=pltpu.CompilerParams(
        use_tc_tiling_on_sc=False, needs_layout_passes=False
    ),
    scratch_types=[
        pltpu.VMEM((_SEQ, _WTOK), jnp.int32),
        [pltpu.VMEM((_WTOK, _EMBED), jnp.float32) for _ in range(_NBUF)],
        [pltpu.VMEM((_NTI, 4, 8, _TB + 1), jnp.float32) for _ in range(_NBUF)],
        pltpu.SemaphoreType.DMA,
        [pltpu.SemaphoreType.DMA for _ in range(_NBUF)],
        [pltpu.SemaphoreType.DMA for _ in range(_NBUF)],
    ],
)(_sc_body)


_VOCAB = 1000000
_CB = 128
_NFULL = _VOCAB // _CB
_TAILR = _VOCAB - _NFULL * _CB
_CPW = _NFULL // _NW
_K1BUF = 2


def _k1_body(tab_hbm, lin_hbm, inv, outv, tailv, gsems, osems):
    wid = lax.axis_index("s") * _NC + lax.axis_index("c")

    iota = lax.iota(jnp.int32, 16)
    rot = [(iota + d) & 15 for d in range(16)]
    eb0 = iota
    eb1 = iota + 16

    def in_copy(c, b):
        return pltpu.make_async_copy(
            tab_hbm.at[:, pl.ds(c * _CB, _CB)], inv[b], gsems[b]
        )

    def out_copy(c, b):
        return pltpu.make_async_copy(
            outv[b], lin_hbm.at[pl.ds(c * 32, 32)], osems[b]
        )

    def transpose_ref(src, b, ncol):
        for ebase in (eb0, eb1):
            @plsc.parallel_loop(0, ncol // 16, 1, unroll=2)
            def _(cb):
                cb16 = cb * 16
                for d in range(16):
                    cvec = rot[d] + cb16
                    v = plsc.load_gather(src, [ebase, cvec])
                    rv = cvec >> 2
                    colv = ((cvec & 3) << 5) + ebase
                    plsc.store_scatter(outv[b], [rv, colv], v)

    def transpose(b, ncol):
        transpose_ref(inv[b], b, ncol)

    def blk(j):
        return wid + j * _NW

    for b in range(_K1BUF):
        in_copy(blk(b), b).start()

    def step(j, b, issue, wait_out):
        c = blk(j)
        in_copy(c, b).wait()
        if wait_out:
            out_copy(blk(j - _K1BUF), b).wait()
        transpose(b, _CB)
        out_copy(c, b).start()
        if issue:
            in_copy(blk(j + _K1BUF), b).start()

    for j in range(_K1BUF):
        step(j, j, True, False)

    @pl.loop(_K1BUF, _CPW - _K1BUF, step=_K1BUF)
    def _(jj):
        for u in range(_K1BUF):
            step(jj + u, u, True, True)

    for j in range(_CPW - _K1BUF, _CPW):
        step(j, j % _K1BUF, False, True)
    for j in range(_CPW - _K1BUF, _CPW):
        out_copy(blk(j), j % _K1BUF).wait()

    @pl.when(wid < 4)
    def _():
        c = _CPW * _NW + wid
        in_copy(c, 0).start()
        in_copy(c, 0).wait()
        transpose(0, _CB)
        cp = out_copy(c, 0)
        cp.start()
        cp.wait()

    @pl.when(wid == 4)
    def _():
        cp = pltpu.make_async_copy(
            tab_hbm.at[:, pl.ds(_NFULL * _CB, _TAILR)], tailv, gsems[0]
        )
        cp.start()
        cp.wait()
        transpose_ref(tailv, 0, _TAILR)
        cp = pltpu.make_async_copy(
            outv[0].at[pl.ds(0, _TAILR // 4)],
            lin_hbm.at[pl.ds(_NFULL * 32, _TAILR // 4)],
            osems[0],
        )
        cp.start()
        cp.wait()


_k1_detile = functools.partial(
    pl.kernel,
    out_type=jax.ShapeDtypeStruct((_VOCAB // 4, _CB), jnp.float32),
    mesh=plsc.VectorSubcoreMesh(core_axis_name="c", subcore_axis_name="s"),
    compiler_params=pltpu.CompilerParams(
        use_tc_tiling_on_sc=True, needs_layout_passes=False
    ),
    scratch_types=[
        [pltpu.VMEM((_EMBED, _CB), jnp.float32) for _ in range(_K1BUF)],
        [pltpu.VMEM((_EMBED, _CB), jnp.float32) for _ in range(_K1BUF)],
        pltpu.VMEM((_EMBED, _TAILR), jnp.float32),
        [pltpu.SemaphoreType.DMA for _ in range(_K1BUF)],
        [pltpu.SemaphoreType.DMA for _ in range(_K1BUF)],
    ],
)(_k1_body)


@jax.jit
def kernel(tokenid, table):
    idx_t = jnp.swapaxes(tokenid, 0, 1)
    lin = _k1_detile(jnp.swapaxes(table, 0, 1))
    table_lin = lin.reshape(_VOCAB, _EMBED)
    out5 = _sc_gather(table_lin, idx_t)
    return out5.transpose(2, 4, 0, 1, 3).reshape(_NTOK, _SEQ, _EMBED)
